# baseline (speedup 1.0000x reference)
import jax
import jax.numpy as jnp
from jax.experimental import pallas as pl
from jax.experimental.pallas import tpu as pltpu


def kernel(atom_mask, W):
    B, S, A = atom_mask.shape
    D = W.shape[1]
    N = B * S
    T = 256
    m2 = atom_mask.reshape(N, A)

    def body(m_ref, w_ref, o_ref):
        o_ref[...] = m_ref[...][:, :, None] * w_ref[...][None, :, :]

    out = pl.pallas_call(
        body,
        grid=(N // T,),
        in_specs=[
            pl.BlockSpec((T, A), lambda i: (i, 0)),
            pl.BlockSpec((A, D), lambda i: (0, 0)),
        ],
        out_specs=pl.BlockSpec((T, A, D), lambda i: (i, 0, 0)),
        out_shape=jax.ShapeDtypeStruct((N, A, D), jnp.float32),
        compiler_params=pltpu.CompilerParams(
            dimension_semantics=("arbitrary",),
        ),
    )(m2, W)
    return out.reshape(B, S, A, D)
